# baseline (speedup 1.0000x reference)
import functools

import jax
import jax.numpy as jnp
from jax import lax
from jax.experimental import pallas as pl
from jax.experimental.pallas import tpu as pltpu
from jax.experimental.pallas import tpu_sc as plsc


def _diag_sc_kernel(n, nadj_hbm, eadj_hbm, dn_hbm, de_hbm, idx_v, buf_n, buf_e, sem):
    info = plsc.get_sparse_core_info()
    nw = info.num_cores * info.num_subcores
    per = n // nw
    wid = lax.axis_index("s") * info.num_cores + lax.axis_index("c")
    base = wid * per
    for j in range(per // 16):
        lane = lax.iota(jnp.int32, 16)
        idx_v[pl.ds(j * 16, 16)] = (base + j * 16 + lane) * (n + 1)
    pltpu.async_copy(nadj_hbm.at[idx_v], buf_n, sem).wait()
    pltpu.async_copy(eadj_hbm.at[idx_v], buf_e, sem).wait()
    pltpu.sync_copy(buf_n, dn_hbm.at[pl.ds(base, per)])
    pltpu.sync_copy(buf_e, de_hbm.at[pl.ds(base, per)])


def _main_kernel(nadj_ref, eadj_ref, feats_ref, nw_ref, ew_ref, nb_ref, eb_ref,
                 dn_ref, de_ref, o_ref, acc_ref, y_ref):
    i = pl.program_id(0)
    k = pl.program_id(1)
    nk = pl.num_programs(1)
    out = o_ref.shape[1]
    bm, bk = nadj_ref.shape

    @pl.when(jnp.logical_and(i == 0, k == 0))
    def _build_y():
        f = feats_ref[...]
        in_dim = f.shape[1]
        y1 = jnp.dot(f, nw_ref[:in_dim, :], preferred_element_type=jnp.float32)
        y2 = jnp.dot(f, nw_ref[in_dim:, :], preferred_element_type=jnp.float32)
        y3 = jnp.dot(f, ew_ref[...], preferred_element_type=jnp.float32)
        y_ref[:, :out] = y1.astype(jnp.bfloat16)
        y_ref[:, out:2 * out] = y2.astype(jnp.bfloat16)
        y_ref[:, 2 * out:] = y3.astype(jnp.bfloat16)

    @pl.when(k == 0)
    def _init():
        dn = (((0,), (0,)), ((), ()))
        acc_ref[...] = (
            lax.dot_general(de_ref[...], nb_ref[...], dn,
                            preferred_element_type=jnp.float32)
            + lax.dot_general(dn_ref[...], eb_ref[...], dn,
                              preferred_element_type=jnp.float32))

    a_n = nadj_ref[...]
    a_e = eadj_ref[...]
    pos = (a_n > 0).astype(jnp.bfloat16)
    neg = (a_n < 0).astype(jnp.bfloat16)
    y = y_ref[pl.ds(k * bk, bk), :]
    acc = jnp.dot(pos, y[:, :out], preferred_element_type=jnp.float32)
    acc += jnp.dot(neg, y[:, out:2 * out], preferred_element_type=jnp.float32)
    acc += jnp.dot(a_e.astype(jnp.bfloat16), y[:, 2 * out:],
                   preferred_element_type=jnp.float32)
    acc_ref[...] += acc

    @pl.when(k == nk - 1)
    def _flush():
        o_ref[...] = acc_ref[...]


@jax.jit
def kernel(feats, node_adj, edge_adj, node_weight, node_bias, edge_weight, edge_bias):
    n, in_dim = feats.shape
    out = node_bias.shape[0]

    diag_n, diag_e = pl.kernel(
        functools.partial(_diag_sc_kernel, n),
        mesh=plsc.VectorSubcoreMesh(core_axis_name="c", subcore_axis_name="s"),
        out_type=(jax.ShapeDtypeStruct((n,), jnp.float32),
                  jax.ShapeDtypeStruct((n,), jnp.float32)),
        scratch_types=[
            pltpu.VMEM((n // 32,), jnp.int32),
            pltpu.VMEM((n // 32,), jnp.float32),
            pltpu.VMEM((n // 32,), jnp.float32),
            pltpu.SemaphoreType.DMA,
        ],
    )(node_adj.reshape(-1), edge_adj.reshape(-1))

    bm = 512
    bk = 4096
    grid = (n // bm, n // bk)

    result = pl.pallas_call(
        _main_kernel,
        grid=grid,
        in_specs=[
            pl.BlockSpec((bm, bk), lambda i, k: (i, k)),
            pl.BlockSpec((bm, bk), lambda i, k: (i, k)),
            pl.BlockSpec((n, in_dim), lambda i, k: (0, 0)),
            pl.BlockSpec((2 * in_dim, out), lambda i, k: (0, 0)),
            pl.BlockSpec((in_dim, out), lambda i, k: (0, 0)),
            pl.BlockSpec((1, out), lambda i, k: (0, 0)),
            pl.BlockSpec((1, out), lambda i, k: (0, 0)),
            pl.BlockSpec((1, bm), lambda i, k: (0, i)),
            pl.BlockSpec((1, bm), lambda i, k: (0, i)),
        ],
        out_specs=pl.BlockSpec((bm, out), lambda i, k: (i, 0)),
        out_shape=jax.ShapeDtypeStruct((n, out), jnp.float32),
        scratch_shapes=[
            pltpu.VMEM((bm, out), jnp.float32),
            pltpu.VMEM((n, 3 * out), jnp.bfloat16),
        ],
        compiler_params=pltpu.CompilerParams(
            dimension_semantics=("arbitrary", "arbitrary"),
        ),
    )(node_adj, edge_adj, feats, node_weight,
      edge_weight, node_bias.reshape(1, out), edge_bias.reshape(1, out),
      diag_n.reshape(1, n), diag_e.reshape(1, n))
    return result


# speedup vs baseline: 3.5850x; 3.5850x over previous
import functools

import jax
import jax.numpy as jnp
from jax.experimental import pallas as pl
from jax.experimental.pallas import tpu as pltpu


def _main_kernel(nadj_ref, eadj_ref, feats_ref, nw_ref, ew_ref, nb_ref, eb_ref,
                 o_ref, acc_ref, y_ref):
    i = pl.program_id(0)
    k = pl.program_id(1)
    nk = pl.num_programs(1)
    out = o_ref.shape[1]
    bm, bk = nadj_ref.shape

    @pl.when(jnp.logical_and(i == 0, k == 0))
    def _build_y():
        f = feats_ref[...]
        in_dim = f.shape[1]
        y1 = jnp.dot(f, nw_ref[:in_dim, :], preferred_element_type=jnp.float32)
        y2 = jnp.dot(f, nw_ref[in_dim:, :], preferred_element_type=jnp.float32)
        y3 = jnp.dot(f, ew_ref[...], preferred_element_type=jnp.float32)
        y_ref[:, :out] = y1.astype(jnp.bfloat16)
        y_ref[:, out:2 * out] = y2.astype(jnp.bfloat16)
        y_ref[:, 2 * out:] = y3.astype(jnp.bfloat16)

    @pl.when(k == 0)
    def _zero():
        acc_ref[...] = jnp.zeros_like(acc_ref)

    a_n = nadj_ref[...]
    a_e = eadj_ref[...]
    pos = (a_n > 0).astype(jnp.bfloat16)
    neg = (a_n < 0).astype(jnp.bfloat16)
    y = y_ref[pl.ds(k * bk, bk), :]
    acc = jnp.dot(pos, y[:, :out], preferred_element_type=jnp.float32)
    acc += jnp.dot(neg, y[:, out:2 * out], preferred_element_type=jnp.float32)
    acc += jnp.dot(a_e.astype(jnp.bfloat16), y[:, 2 * out:],
                   preferred_element_type=jnp.float32)

    @pl.when(jnp.logical_and(i * bm < (k + 1) * bk, k * bk < (i + 1) * bm))
    def _diag():
        col_off = pl.multiple_of(jnp.maximum(i * bm - k * bk, 0), bm)
        m = (jax.lax.broadcasted_iota(jnp.int32, (bm, bm), 0)
             == jax.lax.broadcasted_iota(jnp.int32, (bm, bm), 1))
        sub_e = eadj_ref[:, pl.ds(col_off, bm)]
        sub_n = nadj_ref[:, pl.ds(col_off, bm)]
        diag_e = jnp.sum(jnp.where(m, sub_e, 0.0), axis=1, keepdims=True)
        diag_n = jnp.sum(jnp.where(m, sub_n, 0.0), axis=1, keepdims=True)
        acc_ref[...] += diag_e * nb_ref[...] + diag_n * eb_ref[...]

    acc_ref[...] += acc

    @pl.when(k == nk - 1)
    def _flush():
        o_ref[...] = acc_ref[...]


@jax.jit
def kernel(feats, node_adj, edge_adj, node_weight, node_bias, edge_weight, edge_bias):
    n, in_dim = feats.shape
    out = node_bias.shape[0]

    bm = 512
    bk = 4096
    grid = (n // bm, n // bk)

    result = pl.pallas_call(
        _main_kernel,
        grid=grid,
        in_specs=[
            pl.BlockSpec((bm, bk), lambda i, k: (i, k)),
            pl.BlockSpec((bm, bk), lambda i, k: (i, k)),
            pl.BlockSpec((n, in_dim), lambda i, k: (0, 0)),
            pl.BlockSpec((2 * in_dim, out), lambda i, k: (0, 0)),
            pl.BlockSpec((in_dim, out), lambda i, k: (0, 0)),
            pl.BlockSpec((1, out), lambda i, k: (0, 0)),
            pl.BlockSpec((1, out), lambda i, k: (0, 0)),
        ],
        out_specs=pl.BlockSpec((bm, out), lambda i, k: (i, 0)),
        out_shape=jax.ShapeDtypeStruct((n, out), jnp.float32),
        scratch_shapes=[
            pltpu.VMEM((bm, out), jnp.float32),
            pltpu.VMEM((n, 3 * out), jnp.bfloat16),
        ],
        compiler_params=pltpu.CompilerParams(
            dimension_semantics=("arbitrary", "arbitrary"),
        ),
    )(node_adj, edge_adj, feats, node_weight,
      edge_weight, node_bias.reshape(1, out), edge_bias.reshape(1, out))
    return result
